# baseline (speedup 1.0000x reference)
import jax
import jax.numpy as jnp
from jax.experimental import pallas as pl
from jax.experimental.pallas import tpu as pltpu

V = 100000
N = 1024
SMOOTH = 0.1 / (V - 2)
CONF = 1.0 - 0.1
BR = 16
NG = N // BR
NBUF = 4


def _row_losses(x, t):
    m = jnp.max(x, axis=1, keepdims=True)
    s = jnp.sum(jnp.exp(x - m), axis=1, keepdims=True)
    xsum = jnp.sum(x, axis=1, keepdims=True)
    col = jax.lax.broadcasted_iota(jnp.int32, x.shape, 1)
    tv = jnp.sum(jnp.where(col == t, x, 0.0), axis=1, keepdims=True)
    lse = m + jnp.log(s)
    lp_sum = xsum - V * lse
    lp_t = tv - lse
    loss = -(SMOOTH * (lp_sum - lp_t) + CONF * lp_t)
    mask = (t != 0).astype(jnp.float32)
    return jnp.sum(loss * mask), jnp.sum(mask)


def _body(x_hbm, t_ref, out_ref, *scratch):
    bufs = scratch[:NBUF]
    sems = scratch[NBUF:2 * NBUF]

    def copy(g, b):
        return pltpu.make_async_copy(
            x_hbm.at[pl.ds(g * BR, BR), :], bufs[b], sems[b])

    for b in range(NBUF):
        copy(b, b).start()

    def outer(g0, carry):
        num, den = carry
        for b in range(NBUF):
            g = g0 * NBUF + b
            copy(g, b).wait()
            t = t_ref[pl.ds(g * BR, BR), :]
            nk, dk = _row_losses(bufs[b][...], t)
            num += nk
            den += dk
            nxt = g + NBUF

            @pl.when(nxt < NG)
            def _():
                copy(nxt, b).start()
        return num, den

    num, den = jax.lax.fori_loop(0, NG // NBUF, outer, (0.0, 0.0))
    out_ref[...] = jnp.full((1, 1), num / den, jnp.float32)


def kernel(output, target):
    t = target.astype(jnp.int32).reshape(N, 1)
    out = pl.pallas_call(
        _body,
        in_specs=[pl.BlockSpec(memory_space=pl.ANY),
                  pl.BlockSpec(memory_space=pltpu.VMEM)],
        out_specs=pl.BlockSpec(memory_space=pltpu.VMEM),
        out_shape=jax.ShapeDtypeStruct((1, 1), jnp.float32),
        scratch_shapes=(
            [pltpu.VMEM((BR, V), jnp.float32) for _ in range(NBUF)]
            + [pltpu.SemaphoreType.DMA for _ in range(NBUF)]),
    )(output, t)
    return out.reshape(())


# speedup vs baseline: 1.1040x; 1.1040x over previous
import jax
import jax.numpy as jnp
from jax.experimental import pallas as pl
from jax.experimental.pallas import tpu as pltpu

V = 100000
N = 1024
SMOOTH = 0.1 / (V - 2)
CONF = 1.0 - 0.1
BR = 32
NSTREAM = 2
GRID = N // BR // NSTREAM


def _row_losses(x, t):
    m = jnp.max(x, axis=1, keepdims=True)
    s = jnp.sum(jnp.exp(x - m), axis=1, keepdims=True)
    xsum = jnp.sum(x, axis=1, keepdims=True)
    col = jax.lax.broadcasted_iota(jnp.int32, x.shape, 1)
    tv = jnp.sum(jnp.where(col == t, x, 0.0), axis=1, keepdims=True)
    lse = m + jnp.log(s)
    lp_sum = xsum - V * lse
    lp_t = tv - lse
    loss = -(SMOOTH * (lp_sum - lp_t) + CONF * lp_t)
    mask = (t != 0).astype(jnp.float32)
    return jnp.sum(loss * mask), jnp.sum(mask)


def _body(*refs):
    x_refs = refs[:NSTREAM]
    t_refs = refs[NSTREAM:2 * NSTREAM]
    out_ref = refs[2 * NSTREAM]
    acc_ref = refs[2 * NSTREAM + 1]
    i = pl.program_id(0)

    @pl.when(i == 0)
    def _():
        acc_ref[0] = 0.0
        acc_ref[1] = 0.0

    num = 0.0
    den = 0.0
    for k in range(NSTREAM):
        nk, dk = _row_losses(x_refs[k][...], t_refs[k][...])
        num += nk
        den += dk
    acc_ref[0] += num
    acc_ref[1] += den

    @pl.when(i == GRID - 1)
    def _():
        out_ref[...] = jnp.full((1, 1), acc_ref[0] / acc_ref[1], jnp.float32)


def kernel(output, target):
    t = target.astype(jnp.int32).reshape(N, 1)
    x_specs = [
        pl.BlockSpec((BR, V), lambda i, k=k: (i + k * GRID, 0))
        for k in range(NSTREAM)
    ]
    t_specs = [
        pl.BlockSpec((BR, 1), lambda i, k=k: (i + k * GRID, 0))
        for k in range(NSTREAM)
    ]
    out = pl.pallas_call(
        _body,
        grid=(GRID,),
        in_specs=x_specs + t_specs,
        out_specs=pl.BlockSpec((1, 1), lambda i: (0, 0)),
        out_shape=jax.ShapeDtypeStruct((1, 1), jnp.float32),
        scratch_shapes=[pltpu.SMEM((2,), jnp.float32)],
    )(*([output] * NSTREAM + [t] * NSTREAM))
    return out.reshape(())
